# baseline (speedup 1.0000x reference)
import functools

import jax
import jax.numpy as jnp
from jax import lax
from jax.experimental import pallas as pl
from jax.experimental.pallas import tpu as pltpu
from jax.experimental.pallas import tpu_sc as plsc

N = 10000
E = 320000
DH = 128
DO = 64
G = 64
NC = 2
NS = 16
NPAD = 10240
RT = NPAD // NS

CHUNK = 128
RING = 42
UD = 4
CT = UD * RING
EP = CT * CHUNK * NS

CHUNKC = 78
QC = 69
QT = 4
CTC = QT * QC
EPC = CTC * CHUNKC * NS

_mesh = plsc.VectorSubcoreMesh(core_axis_name="c", subcore_axis_name="s")


@functools.partial(
    pl.kernel,
    out_type=jax.ShapeDtypeStruct((NC * NPAD,), jnp.float32),
    mesh=_mesh,
    scratch_types=[
        pltpu.VMEM((RING, CHUNK), jnp.int32),
        pltpu.VMEM((CHUNK,), jnp.float32),
        pltpu.VMEM((640,), jnp.float32),
        pltpu.VMEM_SHARED((NPAD,), jnp.float32),
        pltpu.SemaphoreType.DMA,
    ],
)
def _deg_kernel(dst_hbm, out_hbm, idx_v, ones_v, zeros_v, acc_sh, sem):
    c = lax.axis_index("c")
    s = lax.axis_index("s")
    for i in range(CHUNK // 16):
        ones_v[pl.ds(i * 16, 16)] = jnp.ones((16,), jnp.float32)

    def zstep(i, _):
        zeros_v[pl.ds(i * 16, 16)] = jnp.zeros((16,), jnp.float32)
        return 0

    lax.fori_loop(0, 640 // 16, zstep, 0)
    pltpu.sync_copy(zeros_v.at[pl.ds(0, RT)], acc_sh.at[pl.ds(s * RT, RT)])
    plsc.subcore_barrier()

    def staging(u, _):
        pltpu.sync_copy(dst_hbm.at[c, s, u], idx_v)

        def pair(i, _):
            j = 2 * i
            d0 = pltpu.async_copy(ones_v, acc_sh.at[idx_v.at[j]],
                                  sem, add=True)
            d1 = pltpu.async_copy(ones_v, acc_sh.at[idx_v.at[j + 1]],
                                  sem, add=True)
            d0.wait()
            d1.wait()
            return 0

        lax.fori_loop(0, RING // 2, pair, 0)
        return 0

    lax.fori_loop(0, UD, staging, 0)
    plsc.subcore_barrier()
    pltpu.sync_copy(acc_sh.at[pl.ds(s * RT, RT)],
                    out_hbm.at[pl.ds(c * NPAD + s * RT, RT)])


@functools.partial(
    pl.kernel,
    out_type=jax.ShapeDtypeStruct((NC, NPAD, DH), jnp.float32),
    mesh=_mesh,
    scratch_types=[
        pltpu.VMEM((QC, CHUNKC), jnp.int32),
        pltpu.VMEM((QC, CHUNKC), jnp.int32),
        pltpu.VMEM((CHUNKC, DH), jnp.float32),
        pltpu.VMEM((CHUNKC, DH), jnp.float32),
        pltpu.VMEM((CHUNKC, DH), jnp.float32),
        pltpu.VMEM_SHARED((NPAD, DH), jnp.float32),
        pltpu.SemaphoreType.DMA,
        pltpu.SemaphoreType.DMA,
        pltpu.SemaphoreType.DMA,
        pltpu.SemaphoreType.DMA,
        pltpu.SemaphoreType.DMA,
        pltpu.SemaphoreType.DMA,
    ],
)
def _conv_kernel(xws_hbm, sd_hbm, out_hbm,
                 srcv, dstv, rb0, rb1, rb2, acc_sh,
                 g0, g1, g2, s0, s1, s2):
    c = lax.axis_index("c")
    s = lax.axis_index("s")
    bufs = (rb0, rb1, rb2)
    gs = (g0, g1, g2)
    ss = (s0, s1, s2)

    def zrow(j, _):
        for k in range(DH // 16):
            rb0[j, pl.ds(k * 16, 16)] = jnp.zeros((16,), jnp.float32)
        return 0

    lax.fori_loop(0, CHUNKC, zrow, 0)
    for t in range(8):
        pltpu.sync_copy(rb0, acc_sh.at[pl.ds(s * RT + t * CHUNKC, CHUNKC)])
    pltpu.sync_copy(rb0.at[pl.ds(0, 16)],
                    acc_sh.at[pl.ds(s * RT + 8 * CHUNKC, 16)])
    plsc.subcore_barrier()

    def staging(q, _):
        pltpu.sync_copy(sd_hbm.at[c, s, q, 0], srcv)
        pltpu.sync_copy(sd_hbm.at[c, s, q, 1], dstv)
        pltpu.async_copy(xws_hbm.at[c].at[srcv.at[0]], rb0, g0)
        pltpu.async_copy(xws_hbm.at[c].at[srcv.at[1]], rb1, g1)

        def body(i, _):
            j = 3 * i
            for k in range(3):
                m = j + k
                B = k
                P = (k + 2) % 3
                pltpu.make_async_copy(
                    xws_hbm.at[c].at[srcv.at[m]], bufs[B], gs[B]).wait()
                pltpu.async_copy(bufs[B], acc_sh.at[dstv.at[m]],
                                 ss[B], add=True)

                @pl.when(m > 0)
                def _():
                    pltpu.make_async_copy(
                        bufs[P], acc_sh.at[dstv.at[m - 1]], ss[P]).wait()

                @pl.when(m + 2 < QC)
                def _():
                    pltpu.async_copy(
                        xws_hbm.at[c].at[srcv.at[m + 2]], bufs[P], gs[P])
            return 0

        lax.fori_loop(0, QC // 3, body, 0)
        pltpu.make_async_copy(bufs[2], acc_sh.at[dstv.at[QC - 1]],
                              ss[2]).wait()
        return 0

    lax.fori_loop(0, QT, staging, 0)
    plsc.subcore_barrier()
    pltpu.sync_copy(acc_sh.at[pl.ds(s * RT, RT)],
                    out_hbm.at[c, pl.ds(s * RT, RT)])


def _mm1_body(x1_ref, x2_ref, w_ref, degp_ref, xws_ref, dinv_ref):
    for b, x_ref in enumerate((x1_ref, x2_ref)):
        deg = degp_ref[b]
        dinv = jnp.where(deg > 0, lax.rsqrt(deg), 0.0)
        dinv_ref[b] = dinv
        xw = jnp.dot(x_ref[...], w_ref[...],
                     preferred_element_type=jnp.float32)
        xws_ref[b] = xw * dinv[:N][:, None]


def _mm1(x1, x2, w, degp):
    return pl.pallas_call(
        _mm1_body,
        out_shape=(jax.ShapeDtypeStruct((NC, N, DH), jnp.float32),
                   jax.ShapeDtypeStruct((NC, NPAD), jnp.float32)),
    )(x1, x2, w, degp)


def _mm2_body(p_ref, dinv_ref, b_ref, w_ref, xws_ref):
    for b in range(2):
        dinv = dinv_ref[b, :N][:, None]
        h = p_ref[b, :N, :] * dinv + b_ref[...]
        xws_ref[b] = jnp.dot(h, w_ref[...],
                             preferred_element_type=jnp.float32) * dinv


def _mm2(p, dinv, bias, w):
    return pl.pallas_call(
        _mm2_body,
        out_shape=jax.ShapeDtypeStruct((NC, N, DH), jnp.float32),
    )(p, dinv, bias, w)


def _head_body(q_ref, dinv_ref, batch1_ref, batch2_ref,
               b2_ref, lw_ref, lb_ref, fw_ref, fb_ref, out_ref):
    def branch(b, batch_ref):
        dinv = dinv_ref[b, :N][:, None]
        h = q_ref[b, :N, :] * dinv + b2_ref[...]
        gids = lax.broadcasted_iota(jnp.int32, (G, N), 0)
        oh = (batch_ref[...][None, :] == gids).astype(jnp.float32)
        sums = jnp.dot(oh, h, preferred_element_type=jnp.float32)
        cnt = jnp.sum(oh, axis=1)
        pooled = sums / jnp.maximum(cnt, 1.0)[:, None]
        a = jnp.maximum(pooled, 0.0)
        a = jnp.maximum(
            jnp.dot(a, lw_ref[...], preferred_element_type=jnp.float32)
            + lb_ref[...], 0.0)
        return (jnp.dot(a, fw_ref[...], preferred_element_type=jnp.float32)
                + fb_ref[...])

    z1 = branch(0, batch1_ref)
    z2 = branch(1, batch2_ref)
    diff = z1 - z2 + 1e-6
    out_ref[...] = jnp.sqrt(jnp.sum(diff * diff, axis=1))


def _head(q, dinv, batch1, batch2, b2, lw, lb, fw, fb):
    return pl.pallas_call(
        _head_body,
        out_shape=jax.ShapeDtypeStruct((G,), jnp.float32),
    )(q, dinv, batch1, batch2, b2, lw, lb, fw, fb)


def _pad_edges_deg(ei):
    npad = EP - E - N
    loop = jnp.arange(N, dtype=jnp.int32)
    pad_dst = N + jnp.arange(npad, dtype=jnp.int32) % (NPAD - N)
    return jnp.concatenate([ei[1], loop, pad_dst]).reshape(NS, UD, RING, CHUNK)


def _pad_edges_conv(ei):
    npad = EPC - E - N
    loop = jnp.arange(N, dtype=jnp.int32)
    pad_src = jnp.arange(npad, dtype=jnp.int32) % N
    pad_dst = N + jnp.arange(npad, dtype=jnp.int32) % (NPAD - N)
    src = jnp.concatenate([ei[0], loop, pad_src]).reshape(
        NS, QT, 1, QC, CHUNKC)
    dst = jnp.concatenate([ei[1], loop, pad_dst]).reshape(
        NS, QT, 1, QC, CHUNKC)
    return jnp.concatenate([src, dst], axis=2)


def kernel(x1, edge_index1, batch1, x2, edge_index2, batch2,
           W1, b1, W2, b2, lin1_W, lin1_b, fin_W, fin_b):
    sd = jnp.stack([_pad_edges_conv(edge_index1),
                    _pad_edges_conv(edge_index2)])
    degp = _deg_kernel(jnp.stack([_pad_edges_deg(edge_index1),
                                  _pad_edges_deg(edge_index2)])
                       ).reshape(NC, NPAD)
    xws, dinv = _mm1(x1, x2, W1, degp)
    p = _conv_kernel(xws, sd)
    xws2 = _mm2(p, dinv, b1, W2)
    q = _conv_kernel(xws2, sd)
    return _head(q, dinv, batch1, batch2, b2, lin1_W, lin1_b, fin_W, fin_b)


# speedup vs baseline: 1.0033x; 1.0033x over previous
import functools

import jax
import jax.numpy as jnp
from jax import lax
from jax.experimental import pallas as pl
from jax.experimental.pallas import tpu as pltpu
from jax.experimental.pallas import tpu_sc as plsc

N = 10000
E = 320000
DH = 128
DO = 64
G = 64
NC = 2
NS = 16
NPAD = 10240
RT = NPAD // NS

CHUNK = 128
RING = 54
UD = 3
CT = UD * RING
EP = CT * CHUNK * NS

CHUNKC = 84
QC = 42
QT = 6
CTC = QT * QC
EPC = CTC * CHUNKC * NS

_mesh = plsc.VectorSubcoreMesh(core_axis_name="c", subcore_axis_name="s")


@functools.partial(
    pl.kernel,
    out_type=jax.ShapeDtypeStruct((NC * NPAD,), jnp.float32),
    mesh=_mesh,
    scratch_types=[
        pltpu.VMEM((RING, CHUNK), jnp.int32),
        pltpu.VMEM((CHUNK,), jnp.float32),
        pltpu.VMEM((640,), jnp.float32),
        pltpu.VMEM_SHARED((NPAD,), jnp.float32),
        pltpu.SemaphoreType.DMA,
    ],
)
def _deg_kernel(dst_hbm, out_hbm, idx_v, ones_v, zeros_v, acc_sh, sem):
    c = lax.axis_index("c")
    s = lax.axis_index("s")
    for i in range(CHUNK // 16):
        ones_v[pl.ds(i * 16, 16)] = jnp.ones((16,), jnp.float32)

    def zstep(i, _):
        zeros_v[pl.ds(i * 16, 16)] = jnp.zeros((16,), jnp.float32)
        return 0

    lax.fori_loop(0, 640 // 16, zstep, 0)
    pltpu.sync_copy(zeros_v.at[pl.ds(0, RT)], acc_sh.at[pl.ds(s * RT, RT)])
    plsc.subcore_barrier()

    def staging(u, _):
        pltpu.sync_copy(dst_hbm.at[c, s, u], idx_v)

        def pair(i, _):
            j = 2 * i
            d0 = pltpu.async_copy(ones_v, acc_sh.at[idx_v.at[j]],
                                  sem, add=True)
            d1 = pltpu.async_copy(ones_v, acc_sh.at[idx_v.at[j + 1]],
                                  sem, add=True)
            d0.wait()
            d1.wait()
            return 0

        lax.fori_loop(0, RING // 2, pair, 0)
        return 0

    lax.fori_loop(0, UD, staging, 0)
    plsc.subcore_barrier()
    pltpu.sync_copy(acc_sh.at[pl.ds(s * RT, RT)],
                    out_hbm.at[pl.ds(c * NPAD + s * RT, RT)])


@functools.partial(
    pl.kernel,
    out_type=jax.ShapeDtypeStruct((NC, NPAD, DH), jnp.float32),
    mesh=_mesh,
    scratch_types=[
        pltpu.VMEM((QC, CHUNKC), jnp.int32),
        pltpu.VMEM((QC, CHUNKC), jnp.int32),
        pltpu.VMEM((CHUNKC, DH), jnp.float32),
        pltpu.VMEM((CHUNKC, DH), jnp.float32),
        pltpu.VMEM((CHUNKC, DH), jnp.float32),
        pltpu.VMEM_SHARED((NPAD, DH), jnp.float32),
        pltpu.SemaphoreType.DMA,
        pltpu.SemaphoreType.DMA,
        pltpu.SemaphoreType.DMA,
        pltpu.SemaphoreType.DMA,
        pltpu.SemaphoreType.DMA,
        pltpu.SemaphoreType.DMA,
    ],
)
def _conv_kernel(xws_hbm, sd_hbm, out_hbm,
                 srcv, dstv, rb0, rb1, rb2, acc_sh,
                 g0, g1, g2, s0, s1, s2):
    c = lax.axis_index("c")
    s = lax.axis_index("s")
    bufs = (rb0, rb1, rb2)
    gs = (g0, g1, g2)
    ss = (s0, s1, s2)

    def zrow(j, _):
        for k in range(DH // 16):
            rb0[j, pl.ds(k * 16, 16)] = jnp.zeros((16,), jnp.float32)
        return 0

    lax.fori_loop(0, CHUNKC, zrow, 0)
    for t in range(7):
        pltpu.sync_copy(rb0, acc_sh.at[pl.ds(s * RT + t * CHUNKC, CHUNKC)])
    pltpu.sync_copy(rb0.at[pl.ds(0, 52)],
                    acc_sh.at[pl.ds(s * RT + 7 * CHUNKC, 52)])
    plsc.subcore_barrier()

    def staging(q, _):
        pltpu.sync_copy(sd_hbm.at[c, s, q, 0], srcv)
        pltpu.sync_copy(sd_hbm.at[c, s, q, 1], dstv)
        pltpu.async_copy(xws_hbm.at[c].at[srcv.at[0]], rb0, g0)
        pltpu.async_copy(xws_hbm.at[c].at[srcv.at[1]], rb1, g1)

        def body(i, _):
            j = 3 * i
            for k in range(3):
                m = j + k
                B = k
                P = (k + 2) % 3
                pltpu.make_async_copy(
                    xws_hbm.at[c].at[srcv.at[m]], bufs[B], gs[B]).wait()
                pltpu.async_copy(bufs[B], acc_sh.at[dstv.at[m]],
                                 ss[B], add=True)

                @pl.when(m > 0)
                def _():
                    pltpu.make_async_copy(
                        bufs[P], acc_sh.at[dstv.at[m - 1]], ss[P]).wait()

                @pl.when(m + 2 < QC)
                def _():
                    pltpu.async_copy(
                        xws_hbm.at[c].at[srcv.at[m + 2]], bufs[P], gs[P])
            return 0

        lax.fori_loop(0, QC // 3, body, 0)
        pltpu.make_async_copy(bufs[2], acc_sh.at[dstv.at[QC - 1]],
                              ss[2]).wait()
        return 0

    lax.fori_loop(0, QT, staging, 0)
    plsc.subcore_barrier()
    pltpu.sync_copy(acc_sh.at[pl.ds(s * RT, RT)],
                    out_hbm.at[c, pl.ds(s * RT, RT)])


def _mm1_body(x1_ref, x2_ref, w_ref, degp_ref, xws_ref, dinv_ref):
    for b, x_ref in enumerate((x1_ref, x2_ref)):
        deg = degp_ref[b]
        dinv = jnp.where(deg > 0, lax.rsqrt(deg), 0.0)
        dinv_ref[b] = dinv
        xw = jnp.dot(x_ref[...], w_ref[...],
                     preferred_element_type=jnp.float32)
        xws_ref[b] = xw * dinv[:N][:, None]


def _mm1(x1, x2, w, degp):
    return pl.pallas_call(
        _mm1_body,
        out_shape=(jax.ShapeDtypeStruct((NC, N, DH), jnp.float32),
                   jax.ShapeDtypeStruct((NC, NPAD), jnp.float32)),
    )(x1, x2, w, degp)


def _mm2_body(p_ref, dinv_ref, b_ref, w_ref, xws_ref):
    for b in range(2):
        dinv = dinv_ref[b, :N][:, None]
        h = p_ref[b, :N, :] * dinv + b_ref[...]
        xws_ref[b] = jnp.dot(h, w_ref[...],
                             preferred_element_type=jnp.float32) * dinv


def _mm2(p, dinv, bias, w):
    return pl.pallas_call(
        _mm2_body,
        out_shape=jax.ShapeDtypeStruct((NC, N, DH), jnp.float32),
    )(p, dinv, bias, w)


def _head_body(q_ref, dinv_ref, batch1_ref, batch2_ref,
               b2_ref, lw_ref, lb_ref, fw_ref, fb_ref, out_ref):
    def branch(b, batch_ref):
        dinv = dinv_ref[b, :N][:, None]
        h = q_ref[b, :N, :] * dinv + b2_ref[...]
        gids = lax.broadcasted_iota(jnp.int32, (G, N), 0)
        oh = (batch_ref[...][None, :] == gids).astype(jnp.float32)
        sums = jnp.dot(oh, h, preferred_element_type=jnp.float32)
        cnt = jnp.sum(oh, axis=1)
        pooled = sums / jnp.maximum(cnt, 1.0)[:, None]
        a = jnp.maximum(pooled, 0.0)
        a = jnp.maximum(
            jnp.dot(a, lw_ref[...], preferred_element_type=jnp.float32)
            + lb_ref[...], 0.0)
        return (jnp.dot(a, fw_ref[...], preferred_element_type=jnp.float32)
                + fb_ref[...])

    z1 = branch(0, batch1_ref)
    z2 = branch(1, batch2_ref)
    diff = z1 - z2 + 1e-6
    out_ref[...] = jnp.sqrt(jnp.sum(diff * diff, axis=1))


def _head(q, dinv, batch1, batch2, b2, lw, lb, fw, fb):
    return pl.pallas_call(
        _head_body,
        out_shape=jax.ShapeDtypeStruct((G,), jnp.float32),
    )(q, dinv, batch1, batch2, b2, lw, lb, fw, fb)


def _pad_edges_deg(ei):
    npad = EP - E - N
    loop = jnp.arange(N, dtype=jnp.int32)
    pad_dst = N + jnp.arange(npad, dtype=jnp.int32) % (NPAD - N)
    return jnp.concatenate([ei[1], loop, pad_dst]).reshape(NS, UD, RING, CHUNK)


def _pad_edges_conv(ei):
    npad = EPC - E - N
    loop = jnp.arange(N, dtype=jnp.int32)
    pad_src = jnp.arange(npad, dtype=jnp.int32) % N
    pad_dst = N + jnp.arange(npad, dtype=jnp.int32) % (NPAD - N)
    src = jnp.concatenate([ei[0], loop, pad_src]).reshape(
        NS, QT, 1, QC, CHUNKC)
    dst = jnp.concatenate([ei[1], loop, pad_dst]).reshape(
        NS, QT, 1, QC, CHUNKC)
    return jnp.concatenate([src, dst], axis=2)


def kernel(x1, edge_index1, batch1, x2, edge_index2, batch2,
           W1, b1, W2, b2, lin1_W, lin1_b, fin_W, fin_b):
    sd = jnp.stack([_pad_edges_conv(edge_index1),
                    _pad_edges_conv(edge_index2)])
    degp = _deg_kernel(jnp.stack([_pad_edges_deg(edge_index1),
                                  _pad_edges_deg(edge_index2)])
                       ).reshape(NC, NPAD)
    xws, dinv = _mm1(x1, x2, W1, degp)
    p = _conv_kernel(xws, sd)
    xws2 = _mm2(p, dinv, b1, W2)
    q = _conv_kernel(xws2, sd)
    return _head(q, dinv, batch1, batch2, b2, lin1_W, lin1_b, fin_W, fin_b)


# speedup vs baseline: 1.0219x; 1.0185x over previous
import functools

import jax
import jax.numpy as jnp
from jax import lax
from jax.experimental import pallas as pl
from jax.experimental.pallas import tpu as pltpu
from jax.experimental.pallas import tpu_sc as plsc

N = 10000
E = 320000
DH = 128
DO = 64
G = 64
NC = 2
NS = 16
NPAD = 10240
RT = NPAD // NS

CHUNK = 128
RING = 54
UD = 3
CT = UD * RING
EP = CT * CHUNK * NS

CHUNKC = 84
QC = 42
QT = 6
CTC = QT * QC
EPC = CTC * CHUNKC * NS

_mesh = plsc.VectorSubcoreMesh(core_axis_name="c", subcore_axis_name="s")


@functools.partial(
    pl.kernel,
    out_type=jax.ShapeDtypeStruct((NC * NPAD,), jnp.float32),
    mesh=_mesh,
    scratch_types=[
        pltpu.VMEM((RING, CHUNK), jnp.int32),
        pltpu.VMEM((CHUNK,), jnp.float32),
        pltpu.VMEM((640,), jnp.float32),
        pltpu.VMEM_SHARED((NPAD,), jnp.float32),
        pltpu.SemaphoreType.DMA,
    ],
)
def _deg_kernel(dst_hbm, out_hbm, idx_v, ones_v, zeros_v, acc_sh, sem):
    c = lax.axis_index("c")
    s = lax.axis_index("s")
    for i in range(CHUNK // 16):
        ones_v[pl.ds(i * 16, 16)] = jnp.ones((16,), jnp.float32)

    def zstep(i, _):
        zeros_v[pl.ds(i * 16, 16)] = jnp.zeros((16,), jnp.float32)
        return 0

    lax.fori_loop(0, 640 // 16, zstep, 0)
    pltpu.sync_copy(zeros_v.at[pl.ds(0, RT)], acc_sh.at[pl.ds(s * RT, RT)])
    plsc.subcore_barrier()

    def staging(u, _):
        pltpu.sync_copy(dst_hbm.at[c, s, u], idx_v)

        def pair(i, _):
            j = 2 * i
            d0 = pltpu.async_copy(ones_v, acc_sh.at[idx_v.at[j]],
                                  sem, add=True)
            d1 = pltpu.async_copy(ones_v, acc_sh.at[idx_v.at[j + 1]],
                                  sem, add=True)
            d0.wait()
            d1.wait()
            return 0

        lax.fori_loop(0, RING // 2, pair, 0)
        return 0

    lax.fori_loop(0, UD, staging, 0)
    plsc.subcore_barrier()
    pltpu.sync_copy(acc_sh.at[pl.ds(s * RT, RT)],
                    out_hbm.at[pl.ds(c * NPAD + s * RT, RT)])


@functools.partial(
    pl.kernel,
    out_type=jax.ShapeDtypeStruct((NC, NPAD, DH), jnp.float32),
    mesh=_mesh,
    scratch_types=[
        pltpu.VMEM((QC, CHUNKC), jnp.int32),
        pltpu.VMEM((QC, CHUNKC), jnp.int32),
        pltpu.VMEM((CHUNKC, DH), jnp.float32),
        pltpu.VMEM((CHUNKC, DH), jnp.float32),
        pltpu.VMEM((CHUNKC, DH), jnp.float32),
        pltpu.VMEM_SHARED((NPAD, DH), jnp.float32),
        pltpu.SemaphoreType.DMA,
        pltpu.SemaphoreType.DMA,
        pltpu.SemaphoreType.DMA,
        pltpu.SemaphoreType.DMA,
        pltpu.SemaphoreType.DMA,
        pltpu.SemaphoreType.DMA,
    ],
)
def _conv_kernel(xws_hbm, sd_hbm, out_hbm,
                 srcv, dstv, rb0, rb1, rb2, acc_sh,
                 g0, g1, g2, s0, s1, s2):
    c = lax.axis_index("c")
    s = lax.axis_index("s")
    bufs = (rb0, rb1, rb2)
    gs = (g0, g1, g2)
    ss = (s0, s1, s2)

    def zrow(j, _):
        for k in range(DH // 16):
            rb0[j, pl.ds(k * 16, 16)] = jnp.zeros((16,), jnp.float32)
        return 0

    lax.fori_loop(0, CHUNKC, zrow, 0)
    for t in range(7):
        pltpu.sync_copy(rb0, acc_sh.at[pl.ds(s * RT + t * CHUNKC, CHUNKC)])
    pltpu.sync_copy(rb0.at[pl.ds(0, 52)],
                    acc_sh.at[pl.ds(s * RT + 7 * CHUNKC, 52)])
    plsc.subcore_barrier()

    def staging(q, _):
        pltpu.sync_copy(sd_hbm.at[c, s, q, 0], srcv)
        pltpu.sync_copy(sd_hbm.at[c, s, q, 1], dstv)
        pltpu.async_copy(xws_hbm.at[c].at[srcv.at[0]], rb0, g0)
        pltpu.async_copy(xws_hbm.at[c].at[srcv.at[1]], rb1, g1)

        def body(i, _):
            j = 3 * i
            for k in range(3):
                m = j + k
                B = k
                P = (k + 2) % 3
                @pl.when(m > 0)
                def _():
                    pltpu.make_async_copy(
                        bufs[P], acc_sh.at[dstv.at[m - 1]], ss[P]).wait()

                @pl.when(m + 2 < QC)
                def _():
                    pltpu.async_copy(
                        xws_hbm.at[c].at[srcv.at[m + 2]], bufs[P], gs[P])

                pltpu.make_async_copy(
                    xws_hbm.at[c].at[srcv.at[m]], bufs[B], gs[B]).wait()
                pltpu.async_copy(bufs[B], acc_sh.at[dstv.at[m]],
                                 ss[B], add=True)
            return 0

        lax.fori_loop(0, QC // 3, body, 0)
        pltpu.make_async_copy(bufs[2], acc_sh.at[dstv.at[QC - 1]],
                              ss[2]).wait()
        return 0

    lax.fori_loop(0, QT, staging, 0)
    plsc.subcore_barrier()
    pltpu.sync_copy(acc_sh.at[pl.ds(s * RT, RT)],
                    out_hbm.at[c, pl.ds(s * RT, RT)])


def _mm1_body(x1_ref, x2_ref, w_ref, degp_ref, xws_ref, dinv_ref):
    for b, x_ref in enumerate((x1_ref, x2_ref)):
        deg = degp_ref[b]
        dinv = jnp.where(deg > 0, lax.rsqrt(deg), 0.0)
        dinv_ref[b] = dinv
        xw = jnp.dot(x_ref[...], w_ref[...],
                     preferred_element_type=jnp.float32)
        xws_ref[b] = xw * dinv[:N][:, None]


def _mm1(x1, x2, w, degp):
    return pl.pallas_call(
        _mm1_body,
        out_shape=(jax.ShapeDtypeStruct((NC, N, DH), jnp.float32),
                   jax.ShapeDtypeStruct((NC, NPAD), jnp.float32)),
    )(x1, x2, w, degp)


def _mm2_body(p_ref, dinv_ref, b_ref, w_ref, xws_ref):
    for b in range(2):
        dinv = dinv_ref[b, :N][:, None]
        h = p_ref[b, :N, :] * dinv + b_ref[...]
        xws_ref[b] = jnp.dot(h, w_ref[...],
                             preferred_element_type=jnp.float32) * dinv


def _mm2(p, dinv, bias, w):
    return pl.pallas_call(
        _mm2_body,
        out_shape=jax.ShapeDtypeStruct((NC, N, DH), jnp.float32),
    )(p, dinv, bias, w)


def _head_body(q_ref, dinv_ref, batch1_ref, batch2_ref,
               b2_ref, lw_ref, lb_ref, fw_ref, fb_ref, out_ref):
    def branch(b, batch_ref):
        dinv = dinv_ref[b, :N][:, None]
        h = q_ref[b, :N, :] * dinv + b2_ref[...]
        gids = lax.broadcasted_iota(jnp.int32, (G, N), 0)
        oh = (batch_ref[...][None, :] == gids).astype(jnp.float32)
        sums = jnp.dot(oh, h, preferred_element_type=jnp.float32)
        cnt = jnp.sum(oh, axis=1)
        pooled = sums / jnp.maximum(cnt, 1.0)[:, None]
        a = jnp.maximum(pooled, 0.0)
        a = jnp.maximum(
            jnp.dot(a, lw_ref[...], preferred_element_type=jnp.float32)
            + lb_ref[...], 0.0)
        return (jnp.dot(a, fw_ref[...], preferred_element_type=jnp.float32)
                + fb_ref[...])

    z1 = branch(0, batch1_ref)
    z2 = branch(1, batch2_ref)
    diff = z1 - z2 + 1e-6
    out_ref[...] = jnp.sqrt(jnp.sum(diff * diff, axis=1))


def _head(q, dinv, batch1, batch2, b2, lw, lb, fw, fb):
    return pl.pallas_call(
        _head_body,
        out_shape=jax.ShapeDtypeStruct((G,), jnp.float32),
    )(q, dinv, batch1, batch2, b2, lw, lb, fw, fb)


def _pad_edges_deg(ei):
    npad = EP - E - N
    loop = jnp.arange(N, dtype=jnp.int32)
    pad_dst = N + jnp.arange(npad, dtype=jnp.int32) % (NPAD - N)
    return jnp.concatenate([ei[1], loop, pad_dst]).reshape(NS, UD, RING, CHUNK)


def _pad_edges_conv(ei):
    npad = EPC - E - N
    loop = jnp.arange(N, dtype=jnp.int32)
    pad_src = jnp.arange(npad, dtype=jnp.int32) % N
    pad_dst = N + jnp.arange(npad, dtype=jnp.int32) % (NPAD - N)
    src = jnp.concatenate([ei[0], loop, pad_src]).reshape(
        NS, QT, 1, QC, CHUNKC)
    dst = jnp.concatenate([ei[1], loop, pad_dst]).reshape(
        NS, QT, 1, QC, CHUNKC)
    return jnp.concatenate([src, dst], axis=2)


def kernel(x1, edge_index1, batch1, x2, edge_index2, batch2,
           W1, b1, W2, b2, lin1_W, lin1_b, fin_W, fin_b):
    sd = jnp.stack([_pad_edges_conv(edge_index1),
                    _pad_edges_conv(edge_index2)])
    degp = _deg_kernel(jnp.stack([_pad_edges_deg(edge_index1),
                                  _pad_edges_deg(edge_index2)])
                       ).reshape(NC, NPAD)
    xws, dinv = _mm1(x1, x2, W1, degp)
    p = _conv_kernel(xws, sd)
    xws2 = _mm2(p, dinv, b1, W2)
    q = _conv_kernel(xws2, sd)
    return _head(q, dinv, batch1, batch2, b2, lin1_W, lin1_b, fin_W, fin_b)
